# baseline (speedup 1.0000x reference)
import jax
import jax.numpy as jnp
from jax import lax
from jax.experimental import pallas as pl
from jax.experimental.pallas import tpu as pltpu
from jax.experimental.pallas import tpu_sc as plsc

N = 10000
D = 128
E = 320000
MU = 0.5
SIGMA = 1.0

N_PAD = 10240
NW = 32
G = 32
NG0 = 640
NG1 = 0
E_PAD = 16 * (NG0 + NG1) * G
IC = 32
RPS = N_PAD // 16
HW = 8


def _build_sc_agg(with_deg: bool):
    mesh = plsc.VectorSubcoreMesh(core_axis_name="c", subcore_axis_name="s")
    out_type = [jax.ShapeDtypeStruct((2, N_PAD, D), jnp.float32)]
    scratch = [
        pltpu.VMEM((IC, G), jnp.int32),
        pltpu.VMEM((IC, G), jnp.int32),
        pltpu.VMEM((G, D), jnp.float32),
        pltpu.VMEM((G, D), jnp.float32),
        pltpu.VMEM((16, D), jnp.float32),
        pltpu.VMEM_SHARED((N_PAD, D), jnp.float32),
        pltpu.SemaphoreType.DMA,
        pltpu.SemaphoreType.DMA,
    ]
    if with_deg:
        out_type.append(jax.ShapeDtypeStruct((2, N_PAD), jnp.float32))
        scratch += [
            pltpu.VMEM((G,), jnp.float32),
            pltpu.VMEM((RPS,), jnp.float32),
            pltpu.VMEM_SHARED((N_PAD,), jnp.float32),
        ]

    def body(x_hbm, src_hbm, dst_hbm, *refs):
        if with_deg:
            (agg_hbm, deg_hbm, src_v, dst_v, rows_a, rows_b, zb_v, acc,
             sem_a, sem_b, ones_v, zd_v, dacc) = refs
        else:
            (agg_hbm, src_v, dst_v, rows_a, rows_b, zb_v, acc,
             sem_a, sem_b) = refs
        cid = lax.axis_index("c")
        sid = lax.axis_index("s")
        gb = (1 - cid) * (sid * NG0) + cid * (16 * NG0 + sid * NG1)
        nout = (1 - cid) * (NG0 // IC) + cid * (NG1 // IC)
        base = sid * RPS

        zeros16 = jnp.zeros((16,), jnp.float32)
        for r in range(16):
            for c in range(D // 16):
                zb_v[r, pl.ds(c * 16, 16)] = zeros16
        if with_deg:
            ones16 = jnp.ones((16,), jnp.float32)
            for r in range(G // 16):
                ones_v[pl.ds(r * 16, 16)] = ones16
            for r in range(RPS // 16):
                zd_v[pl.ds(r * 16, 16)] = zeros16

        def zloop(r, carry):
            pltpu.sync_copy(zb_v, acc.at[pl.ds(base + r * 16, 16), :])
            return carry
        lax.fori_loop(0, RPS // 16, zloop, 0)
        if with_deg:
            pltpu.sync_copy(zd_v, dacc.at[pl.ds(base, RPS)])
        plsc.subcore_barrier()

        def fire(buf, sem, g):
            pltpu.async_copy(x_hbm.at[src_v.at[g]], buf, sem)

        def wait(buf, sem, g):
            pltpu.make_async_copy(x_hbm.at[src_v.at[g]], buf, sem).wait()

        def scat(buf, g):
            pltpu.sync_copy(buf, acc.at[dst_v.at[g]], add=True)
            if with_deg:
                pltpu.sync_copy(ones_v, dacc.at[dst_v.at[g]], add=True)

        def oloop(o, carry):
            pltpu.sync_copy(src_hbm.at[pl.ds(gb + o * IC, IC), :], src_v)
            pltpu.sync_copy(dst_hbm.at[pl.ds(gb + o * IC, IC), :], dst_v)
            fire(rows_a, sem_a, 0)

            def eloop(j, c2):
                ga = 2 * j
                wait(rows_a, sem_a, ga)
                fire(rows_b, sem_b, ga + 1)
                scat(rows_a, ga)
                wait(rows_b, sem_b, ga + 1)

                @pl.when(j < IC // 2 - 1)
                def _():
                    fire(rows_a, sem_a, ga + 2)
                scat(rows_b, ga + 1)
                return c2
            lax.fori_loop(0, IC // 2, eloop, 0)
            return carry
        lax.fori_loop(0, nout, oloop, 0)
        plsc.subcore_barrier()

        pltpu.sync_copy(acc.at[pl.ds(base, RPS), :],
                        agg_hbm.at[cid, pl.ds(base, RPS), :])
        if with_deg:
            pltpu.sync_copy(dacc.at[pl.ds(base, RPS)],
                            deg_hbm.at[cid, pl.ds(base, RPS)])

    return pl.kernel(body, out_type=tuple(out_type), mesh=mesh,
                     scratch_types=scratch)


_SC_AGG_DEG = _build_sc_agg(True)
_SC_AGG = _build_sc_agg(False)

_TCB = 1024


def _tc_layer(x_pad, aggp, degp3, w_s, w_n, b):
    def body(x_ref, a_ref, d_ref, ws_ref, wn_ref, b_ref, o_ref):
        dg = jnp.maximum(d_ref[0] + d_ref[1], 1.0)
        agg = (a_ref[0] + a_ref[1]) / dg
        h = jnp.dot(x_ref[...], ws_ref[...], preferred_element_type=jnp.float32)
        h = h + jnp.dot(agg, wn_ref[...], preferred_element_type=jnp.float32)
        o_ref[...] = jnp.maximum(h + b_ref[...], 0.0)

    return pl.pallas_call(
        body,
        grid=(N_PAD // _TCB,),
        in_specs=[
            pl.BlockSpec((_TCB, D), lambda i: (i, 0)),
            pl.BlockSpec((2, _TCB, D), lambda i: (0, i, 0)),
            pl.BlockSpec((2, _TCB, 1), lambda i: (0, i, 0)),
            pl.BlockSpec((D, D), lambda i: (0, 0)),
            pl.BlockSpec((D, D), lambda i: (0, 0)),
            pl.BlockSpec((1, D), lambda i: (0, 0)),
        ],
        out_specs=pl.BlockSpec((_TCB, D), lambda i: (i, 0)),
        out_shape=jax.ShapeDtypeStruct((N_PAD, D), jnp.float32),
    )(x_pad, aggp, degp3, w_s, w_n, b)


def _tc_layer_head(h1, aggp, degp3, w_s, w_n, b, w_out, b_out):
    def body(x_ref, a_ref, d_ref, ws_ref, wn_ref, b_ref, wo_ref, bo_ref, o_ref):
        dg = jnp.maximum(d_ref[0] + d_ref[1], 1.0)
        agg = (a_ref[0] + a_ref[1]) / dg
        h = jnp.dot(x_ref[...], ws_ref[...], preferred_element_type=jnp.float32)
        h = h + jnp.dot(agg, wn_ref[...], preferred_element_type=jnp.float32)
        h2 = jnp.maximum(h + b_ref[...], 0.0)
        y = jnp.dot(h2, wo_ref[...], preferred_element_type=jnp.float32)
        z = (y + bo_ref[...] - MU) / SIGMA
        o_ref[...] = jnp.exp(-0.5 * z * z)

    return pl.pallas_call(
        body,
        grid=(N_PAD // _TCB,),
        in_specs=[
            pl.BlockSpec((_TCB, D), lambda i: (i, 0)),
            pl.BlockSpec((2, _TCB, D), lambda i: (0, i, 0)),
            pl.BlockSpec((2, _TCB, 1), lambda i: (0, i, 0)),
            pl.BlockSpec((D, D), lambda i: (0, 0)),
            pl.BlockSpec((D, D), lambda i: (0, 0)),
            pl.BlockSpec((1, D), lambda i: (0, 0)),
            pl.BlockSpec((D, HW), lambda i: (0, 0)),
            pl.BlockSpec((1, HW), lambda i: (0, 0)),
        ],
        out_specs=pl.BlockSpec((_TCB, HW), lambda i: (i, 0)),
        out_shape=jax.ShapeDtypeStruct((N_PAD, HW), jnp.float32),
    )(h1, aggp, degp3, w_s, w_n, b, w_out, b_out)


def kernel(x, edge_index, W1_self, W1_nbr, b1, W2_self, W2_nbr, b2, W_out, b_out):
    src = edge_index[0]
    dst = edge_index[1]
    pad_e = E_PAD - E
    srcr = jnp.concatenate(
        [src, jnp.zeros((pad_e,), jnp.int32)]).reshape(E_PAD // G, G)
    dstr = jnp.concatenate(
        [dst, jnp.full((pad_e,), N_PAD - 1, jnp.int32)]).reshape(E_PAD // G, G)
    x_pad = jnp.pad(x, ((0, N_PAD - N), (0, 0)))

    agg1p, degp = _SC_AGG_DEG(x_pad, srcr, dstr)
    degp3 = degp.reshape(2, N_PAD, 1)
    h1 = _tc_layer(x_pad, agg1p, degp3, W1_self, W1_nbr, b1.reshape(1, D))
    (agg2p,) = _SC_AGG(h1, srcr, dstr)
    out_w = _tc_layer_head(
        h1, agg2p, degp3, W2_self, W2_nbr, b2.reshape(1, D),
        jnp.pad(W_out, ((0, 0), (0, HW - 1))),
        jnp.pad(b_out, (0, HW - 1)).reshape(1, HW))
    return out_w[:N, :1]


# speedup vs baseline: 1.7259x; 1.7259x over previous
import jax
import jax.numpy as jnp
from jax import lax
from jax.experimental import pallas as pl
from jax.experimental.pallas import tpu as pltpu
from jax.experimental.pallas import tpu_sc as plsc

N = 10000
D = 128
E = 320000
MU = 0.5
SIGMA = 1.0

N_PAD = 10240
NW = 32
G = 32
NG0 = 480
NG1 = 160
E_PAD = 16 * (NG0 + NG1) * G
IC = 32
RPS = N_PAD // 16
HW = 8


def _build_sc_agg(with_deg: bool):
    mesh = plsc.VectorSubcoreMesh(core_axis_name="c", subcore_axis_name="s")
    out_type = [jax.ShapeDtypeStruct((2, N_PAD, D), jnp.float32)]
    scratch = [
        pltpu.VMEM((IC, G), jnp.int32),
        pltpu.VMEM((IC, G), jnp.int32),
        pltpu.VMEM((G, D), jnp.float32),
        pltpu.VMEM((G, D), jnp.float32),
        pltpu.VMEM((16, D), jnp.float32),
        pltpu.VMEM_SHARED((N_PAD, D), jnp.float32),
        pltpu.SemaphoreType.DMA,
        pltpu.SemaphoreType.DMA,
    ]
    if with_deg:
        out_type.append(jax.ShapeDtypeStruct((2, N_PAD), jnp.float32))
        scratch += [
            pltpu.VMEM((G,), jnp.float32),
            pltpu.VMEM((RPS,), jnp.float32),
            pltpu.VMEM_SHARED((N_PAD,), jnp.float32),
        ]

    def body(x_hbm, src_hbm, dst_hbm, *refs):
        if with_deg:
            (agg_hbm, deg_hbm, src_v, dst_v, rows_a, rows_b, zb_v, acc,
             sem_a, sem_b, ones_v, zd_v, dacc) = refs
        else:
            (agg_hbm, src_v, dst_v, rows_a, rows_b, zb_v, acc,
             sem_a, sem_b) = refs
        cid = lax.axis_index("c")
        sid = lax.axis_index("s")
        gb = (1 - cid) * (sid * NG0) + cid * (16 * NG0 + sid * NG1)
        nout = (1 - cid) * (NG0 // IC) + cid * (NG1 // IC)
        base = sid * RPS

        zeros16 = jnp.zeros((16,), jnp.float32)
        for r in range(16):
            for c in range(D // 16):
                zb_v[r, pl.ds(c * 16, 16)] = zeros16
        if with_deg:
            ones16 = jnp.ones((16,), jnp.float32)
            for r in range(G // 16):
                ones_v[pl.ds(r * 16, 16)] = ones16
            for r in range(RPS // 16):
                zd_v[pl.ds(r * 16, 16)] = zeros16

        def zloop(r, carry):
            pltpu.sync_copy(zb_v, acc.at[pl.ds(base + r * 16, 16), :])
            return carry
        lax.fori_loop(0, RPS // 16, zloop, 0)
        if with_deg:
            pltpu.sync_copy(zd_v, dacc.at[pl.ds(base, RPS)])
        plsc.subcore_barrier()

        def fire(buf, sem, g):
            pltpu.async_copy(x_hbm.at[src_v.at[g]], buf, sem)

        def wait(buf, sem, g):
            pltpu.make_async_copy(x_hbm.at[src_v.at[g]], buf, sem).wait()

        def scat(buf, g):
            pltpu.sync_copy(buf, acc.at[dst_v.at[g]], add=True)
            if with_deg:
                pltpu.sync_copy(ones_v, dacc.at[dst_v.at[g]], add=True)

        def oloop(o, carry):
            pltpu.sync_copy(src_hbm.at[pl.ds(gb + o * IC, IC), :], src_v)
            pltpu.sync_copy(dst_hbm.at[pl.ds(gb + o * IC, IC), :], dst_v)
            fire(rows_a, sem_a, 0)

            def eloop(j, c2):
                ga = 2 * j
                wait(rows_a, sem_a, ga)
                fire(rows_b, sem_b, ga + 1)
                scat(rows_a, ga)
                wait(rows_b, sem_b, ga + 1)

                @pl.when(j < IC // 2 - 1)
                def _():
                    fire(rows_a, sem_a, ga + 2)
                scat(rows_b, ga + 1)
                return c2
            lax.fori_loop(0, IC // 2, eloop, 0)
            return carry
        lax.fori_loop(0, nout, oloop, 0)
        plsc.subcore_barrier()

        pltpu.sync_copy(acc.at[pl.ds(base, RPS), :],
                        agg_hbm.at[cid, pl.ds(base, RPS), :])
        if with_deg:
            pltpu.sync_copy(dacc.at[pl.ds(base, RPS)],
                            deg_hbm.at[cid, pl.ds(base, RPS)])

    return pl.kernel(body, out_type=tuple(out_type), mesh=mesh,
                     scratch_types=scratch)


_SC_AGG_DEG = _build_sc_agg(True)
_SC_AGG = _build_sc_agg(False)

_TCB = 1024


def _tc_layer(x_pad, aggp, degp3, w_s, w_n, b):
    def body(x_ref, a_ref, d_ref, ws_ref, wn_ref, b_ref, o_ref):
        dg = jnp.maximum(d_ref[0] + d_ref[1], 1.0)
        agg = (a_ref[0] + a_ref[1]) / dg
        h = jnp.dot(x_ref[...], ws_ref[...], preferred_element_type=jnp.float32)
        h = h + jnp.dot(agg, wn_ref[...], preferred_element_type=jnp.float32)
        o_ref[...] = jnp.maximum(h + b_ref[...], 0.0)

    return pl.pallas_call(
        body,
        grid=(N_PAD // _TCB,),
        in_specs=[
            pl.BlockSpec((_TCB, D), lambda i: (i, 0)),
            pl.BlockSpec((2, _TCB, D), lambda i: (0, i, 0)),
            pl.BlockSpec((2, _TCB, 1), lambda i: (0, i, 0)),
            pl.BlockSpec((D, D), lambda i: (0, 0)),
            pl.BlockSpec((D, D), lambda i: (0, 0)),
            pl.BlockSpec((1, D), lambda i: (0, 0)),
        ],
        out_specs=pl.BlockSpec((_TCB, D), lambda i: (i, 0)),
        out_shape=jax.ShapeDtypeStruct((N_PAD, D), jnp.float32),
    )(x_pad, aggp, degp3, w_s, w_n, b)


def _tc_layer_head(h1, aggp, degp3, w_s, w_n, b, w_out, b_out):
    def body(x_ref, a_ref, d_ref, ws_ref, wn_ref, b_ref, wo_ref, bo_ref, o_ref):
        dg = jnp.maximum(d_ref[0] + d_ref[1], 1.0)
        agg = (a_ref[0] + a_ref[1]) / dg
        h = jnp.dot(x_ref[...], ws_ref[...], preferred_element_type=jnp.float32)
        h = h + jnp.dot(agg, wn_ref[...], preferred_element_type=jnp.float32)
        h2 = jnp.maximum(h + b_ref[...], 0.0)
        y = jnp.dot(h2, wo_ref[...], preferred_element_type=jnp.float32)
        z = (y + bo_ref[...] - MU) / SIGMA
        o_ref[...] = jnp.exp(-0.5 * z * z)

    return pl.pallas_call(
        body,
        grid=(N_PAD // _TCB,),
        in_specs=[
            pl.BlockSpec((_TCB, D), lambda i: (i, 0)),
            pl.BlockSpec((2, _TCB, D), lambda i: (0, i, 0)),
            pl.BlockSpec((2, _TCB, 1), lambda i: (0, i, 0)),
            pl.BlockSpec((D, D), lambda i: (0, 0)),
            pl.BlockSpec((D, D), lambda i: (0, 0)),
            pl.BlockSpec((1, D), lambda i: (0, 0)),
            pl.BlockSpec((D, HW), lambda i: (0, 0)),
            pl.BlockSpec((1, HW), lambda i: (0, 0)),
        ],
        out_specs=pl.BlockSpec((_TCB, HW), lambda i: (i, 0)),
        out_shape=jax.ShapeDtypeStruct((N_PAD, HW), jnp.float32),
    )(h1, aggp, degp3, w_s, w_n, b, w_out, b_out)


def kernel(x, edge_index, W1_self, W1_nbr, b1, W2_self, W2_nbr, b2, W_out, b_out):
    src = edge_index[0]
    dst = edge_index[1]
    pad_e = E_PAD - E
    srcr = jnp.concatenate(
        [src, jnp.zeros((pad_e,), jnp.int32)]).reshape(E_PAD // G, G)
    dstr = jnp.concatenate(
        [dst, jnp.full((pad_e,), N_PAD - 1, jnp.int32)]).reshape(E_PAD // G, G)
    x_pad = jnp.pad(x, ((0, N_PAD - N), (0, 0)))

    agg1p, degp = _SC_AGG_DEG(x_pad, srcr, dstr)
    degp3 = degp.reshape(2, N_PAD, 1)
    h1 = _tc_layer(x_pad, agg1p, degp3, W1_self, W1_nbr, b1.reshape(1, D))
    (agg2p,) = _SC_AGG(h1, srcr, dstr)
    out_w = _tc_layer_head(
        h1, agg2p, degp3, W2_self, W2_nbr, b2.reshape(1, D),
        jnp.pad(W_out, ((0, 0), (0, HW - 1))),
        jnp.pad(b_out, (0, HW - 1)).reshape(1, HW))
    return out_w[:N, :1]


# speedup vs baseline: 1.8128x; 1.0503x over previous
import jax
import jax.numpy as jnp
from jax import lax
from jax.experimental import pallas as pl
from jax.experimental.pallas import tpu as pltpu
from jax.experimental.pallas import tpu_sc as plsc

N = 10000
D = 128
E = 320000
MU = 0.5
SIGMA = 1.0

N_PAD = 10240
NW = 32
G = 32
NG0 = 480
NG1 = 160
E_PAD = 16 * (NG0 + NG1) * G
IC = 32
RPS = N_PAD // 16
HW = 8


def _build_sc_agg(with_deg: bool):
    mesh = plsc.VectorSubcoreMesh(core_axis_name="c", subcore_axis_name="s")
    out_type = [jax.ShapeDtypeStruct((2, N_PAD, D), jnp.float32)]
    scratch = [
        pltpu.VMEM((IC, G), jnp.int32),
        pltpu.VMEM((IC, G), jnp.int32),
        pltpu.VMEM((G, D), jnp.float32),
        pltpu.VMEM((G, D), jnp.float32),
        pltpu.VMEM((G, D), jnp.float32),
        pltpu.VMEM((G, D), jnp.float32),
        pltpu.VMEM((16, D), jnp.float32),
        pltpu.VMEM_SHARED((N_PAD, D), jnp.float32),
        pltpu.SemaphoreType.DMA,
        pltpu.SemaphoreType.DMA,
        pltpu.SemaphoreType.DMA,
        pltpu.SemaphoreType.DMA,
    ]
    if with_deg:
        out_type.append(jax.ShapeDtypeStruct((2, N_PAD), jnp.float32))
        scratch += [
            pltpu.VMEM((G,), jnp.float32),
            pltpu.VMEM((RPS,), jnp.float32),
            pltpu.VMEM_SHARED((N_PAD,), jnp.float32),
        ]

    def body(x_hbm, src_hbm, dst_hbm, *refs):
        if with_deg:
            (agg_hbm, deg_hbm, src_v, dst_v, r0, r1, r2, r3, zb_v,
             acc, s0, s1, s2, s3, ones_v, zd_v, dacc) = refs
        else:
            (agg_hbm, src_v, dst_v, r0, r1, r2, r3, zb_v, acc,
             s0, s1, s2, s3) = refs
        rows = (r0, r1, r2, r3)
        sems = (s0, s1, s2, s3)
        cid = lax.axis_index("c")
        sid = lax.axis_index("s")
        gb = (1 - cid) * (sid * NG0) + cid * (16 * NG0 + sid * NG1)
        nout = (1 - cid) * (NG0 // IC) + cid * (NG1 // IC)
        base = sid * RPS

        zeros16 = jnp.zeros((16,), jnp.float32)
        for r in range(16):
            for c in range(D // 16):
                zb_v[r, pl.ds(c * 16, 16)] = zeros16
        if with_deg:
            ones16 = jnp.ones((16,), jnp.float32)
            for r in range(G // 16):
                ones_v[pl.ds(r * 16, 16)] = ones16
            for r in range(RPS // 16):
                zd_v[pl.ds(r * 16, 16)] = zeros16

        def zloop(r, carry):
            pltpu.sync_copy(zb_v, acc.at[pl.ds(base + r * 16, 16), :])
            return carry
        lax.fori_loop(0, RPS // 16, zloop, 0)
        if with_deg:
            pltpu.sync_copy(zd_v, dacc.at[pl.ds(base, RPS)])
        plsc.subcore_barrier()

        def fire(b, g):
            pltpu.async_copy(x_hbm.at[src_v.at[g]], rows[b], sems[b])

        def wait(b, g):
            pltpu.make_async_copy(
                x_hbm.at[src_v.at[g]], rows[b], sems[b]).wait()

        def scat(b, g):
            pltpu.sync_copy(rows[b], acc.at[dst_v.at[g]], add=True)
            if with_deg:
                pltpu.sync_copy(ones_v, dacc.at[dst_v.at[g]], add=True)

        def oloop(o, carry):
            pltpu.sync_copy(src_hbm.at[pl.ds(gb + o * IC, IC), :], src_v)
            pltpu.sync_copy(dst_hbm.at[pl.ds(gb + o * IC, IC), :], dst_v)
            for b in range(3):
                fire(b, b)

            def eloop(j, c2):
                g4 = 4 * j
                for b in range(4):
                    g = g4 + b

                    @pl.when(g + 3 < IC)
                    def _(b=b, g=g):
                        fire((b + 3) % 4, g + 3)
                    wait(b, g)
                    scat(b, g)
                return c2
            lax.fori_loop(0, IC // 4, eloop, 0)
            return carry
        lax.fori_loop(0, nout, oloop, 0)
        plsc.subcore_barrier()

        pltpu.sync_copy(acc.at[pl.ds(base, RPS), :],
                        agg_hbm.at[cid, pl.ds(base, RPS), :])
        if with_deg:
            pltpu.sync_copy(dacc.at[pl.ds(base, RPS)],
                            deg_hbm.at[cid, pl.ds(base, RPS)])

    return pl.kernel(body, out_type=tuple(out_type), mesh=mesh,
                     scratch_types=scratch)


_SC_AGG_DEG = _build_sc_agg(True)
_SC_AGG = _build_sc_agg(False)

_TCB = 1024


def _tc_layer(x_pad, aggp, degp3, w_s, w_n, b):
    def body(x_ref, a_ref, d_ref, ws_ref, wn_ref, b_ref, o_ref):
        dg = jnp.maximum(d_ref[0] + d_ref[1], 1.0)
        agg = (a_ref[0] + a_ref[1]) / dg
        h = jnp.dot(x_ref[...], ws_ref[...], preferred_element_type=jnp.float32)
        h = h + jnp.dot(agg, wn_ref[...], preferred_element_type=jnp.float32)
        o_ref[...] = jnp.maximum(h + b_ref[...], 0.0)

    return pl.pallas_call(
        body,
        grid=(N_PAD // _TCB,),
        in_specs=[
            pl.BlockSpec((_TCB, D), lambda i: (i, 0)),
            pl.BlockSpec((2, _TCB, D), lambda i: (0, i, 0)),
            pl.BlockSpec((2, _TCB, 1), lambda i: (0, i, 0)),
            pl.BlockSpec((D, D), lambda i: (0, 0)),
            pl.BlockSpec((D, D), lambda i: (0, 0)),
            pl.BlockSpec((1, D), lambda i: (0, 0)),
        ],
        out_specs=pl.BlockSpec((_TCB, D), lambda i: (i, 0)),
        out_shape=jax.ShapeDtypeStruct((N_PAD, D), jnp.float32),
    )(x_pad, aggp, degp3, w_s, w_n, b)


def _tc_layer_head(h1, aggp, degp3, w_s, w_n, b, w_out, b_out):
    def body(x_ref, a_ref, d_ref, ws_ref, wn_ref, b_ref, wo_ref, bo_ref, o_ref):
        dg = jnp.maximum(d_ref[0] + d_ref[1], 1.0)
        agg = (a_ref[0] + a_ref[1]) / dg
        h = jnp.dot(x_ref[...], ws_ref[...], preferred_element_type=jnp.float32)
        h = h + jnp.dot(agg, wn_ref[...], preferred_element_type=jnp.float32)
        h2 = jnp.maximum(h + b_ref[...], 0.0)
        y = jnp.dot(h2, wo_ref[...], preferred_element_type=jnp.float32)
        z = (y + bo_ref[...] - MU) / SIGMA
        o_ref[...] = jnp.exp(-0.5 * z * z)

    return pl.pallas_call(
        body,
        grid=(N_PAD // _TCB,),
        in_specs=[
            pl.BlockSpec((_TCB, D), lambda i: (i, 0)),
            pl.BlockSpec((2, _TCB, D), lambda i: (0, i, 0)),
            pl.BlockSpec((2, _TCB, 1), lambda i: (0, i, 0)),
            pl.BlockSpec((D, D), lambda i: (0, 0)),
            pl.BlockSpec((D, D), lambda i: (0, 0)),
            pl.BlockSpec((1, D), lambda i: (0, 0)),
            pl.BlockSpec((D, HW), lambda i: (0, 0)),
            pl.BlockSpec((1, HW), lambda i: (0, 0)),
        ],
        out_specs=pl.BlockSpec((_TCB, HW), lambda i: (i, 0)),
        out_shape=jax.ShapeDtypeStruct((N_PAD, HW), jnp.float32),
    )(h1, aggp, degp3, w_s, w_n, b, w_out, b_out)


def kernel(x, edge_index, W1_self, W1_nbr, b1, W2_self, W2_nbr, b2, W_out, b_out):
    src = edge_index[0]
    dst = edge_index[1]
    pad_e = E_PAD - E
    srcr = jnp.concatenate(
        [src, jnp.zeros((pad_e,), jnp.int32)]).reshape(E_PAD // G, G)
    dstr = jnp.concatenate(
        [dst, jnp.full((pad_e,), N_PAD - 1, jnp.int32)]).reshape(E_PAD // G, G)
    x_pad = jnp.pad(x, ((0, N_PAD - N), (0, 0)))

    agg1p, degp = _SC_AGG_DEG(x_pad, srcr, dstr)
    degp3 = degp.reshape(2, N_PAD, 1)
    h1 = _tc_layer(x_pad, agg1p, degp3, W1_self, W1_nbr, b1.reshape(1, D))
    (agg2p,) = _SC_AGG(h1, srcr, dstr)
    out_w = _tc_layer_head(
        h1, agg2p, degp3, W2_self, W2_nbr, b2.reshape(1, D),
        jnp.pad(W_out, ((0, 0), (0, HW - 1))),
        jnp.pad(b_out, (0, HW - 1)).reshape(1, HW))
    return out_w[:N, :1]


# speedup vs baseline: 1.9520x; 1.0768x over previous
import jax
import jax.numpy as jnp
from jax import lax
from jax.experimental import pallas as pl
from jax.experimental.pallas import tpu as pltpu
from jax.experimental.pallas import tpu_sc as plsc

N = 10000
D = 128
E = 320000
MU = 0.5
SIGMA = 1.0

N_PAD = 10240
NW = 32
G = 32
NG0 = 544
NG1 = 96
E_PAD = 16 * (NG0 + NG1) * G
IC = 32
RPS = N_PAD // 16
HW = 8


def _build_sc_agg(with_deg: bool):
    mesh = plsc.VectorSubcoreMesh(core_axis_name="c", subcore_axis_name="s")
    out_type = [jax.ShapeDtypeStruct((2, N_PAD, D), jnp.float32)]
    scratch = [
        pltpu.VMEM((IC, G), jnp.int32),
        pltpu.VMEM((IC, G), jnp.int32),
        pltpu.VMEM((G, D), jnp.float32),
        pltpu.VMEM((G, D), jnp.float32),
        pltpu.VMEM((G, D), jnp.float32),
        pltpu.VMEM((G, D), jnp.float32),
        pltpu.VMEM((16, D), jnp.float32),
        pltpu.VMEM_SHARED((N_PAD, D), jnp.float32),
        pltpu.SemaphoreType.DMA,
        pltpu.SemaphoreType.DMA,
        pltpu.SemaphoreType.DMA,
        pltpu.SemaphoreType.DMA,
    ]
    if with_deg:
        out_type.append(jax.ShapeDtypeStruct((2, N_PAD), jnp.float32))
        scratch += [
            pltpu.VMEM((G,), jnp.float32),
            pltpu.VMEM((RPS,), jnp.float32),
            pltpu.VMEM_SHARED((N_PAD,), jnp.float32),
        ]

    def body(x_hbm, src_hbm, dst_hbm, *refs):
        if with_deg:
            (agg_hbm, deg_hbm, src_v, dst_v, r0, r1, r2, r3, zb_v,
             acc, s0, s1, s2, s3, ones_v, zd_v, dacc) = refs
        else:
            (agg_hbm, src_v, dst_v, r0, r1, r2, r3, zb_v, acc,
             s0, s1, s2, s3) = refs
        rows = (r0, r1, r2, r3)
        sems = (s0, s1, s2, s3)
        cid = lax.axis_index("c")
        sid = lax.axis_index("s")
        gb = (1 - cid) * (sid * NG0) + cid * (16 * NG0 + sid * NG1)
        nout = (1 - cid) * (NG0 // IC) + cid * (NG1 // IC)
        base = sid * RPS

        zeros16 = jnp.zeros((16,), jnp.float32)
        for r in range(16):
            for c in range(D // 16):
                zb_v[r, pl.ds(c * 16, 16)] = zeros16
        if with_deg:
            ones16 = jnp.ones((16,), jnp.float32)
            for r in range(G // 16):
                ones_v[pl.ds(r * 16, 16)] = ones16
            for r in range(RPS // 16):
                zd_v[pl.ds(r * 16, 16)] = zeros16

        def zloop(r, carry):
            pltpu.sync_copy(zb_v, acc.at[pl.ds(base + r * 16, 16), :])
            return carry
        lax.fori_loop(0, RPS // 16, zloop, 0)
        if with_deg:
            pltpu.sync_copy(zd_v, dacc.at[pl.ds(base, RPS)])
        plsc.subcore_barrier()

        def fire(b, g):
            pltpu.async_copy(x_hbm.at[src_v.at[g]], rows[b], sems[b])

        def wait(b, g):
            pltpu.make_async_copy(
                x_hbm.at[src_v.at[g]], rows[b], sems[b]).wait()

        def scat(b, g):
            pltpu.sync_copy(rows[b], acc.at[dst_v.at[g]], add=True)
            if with_deg:
                pltpu.sync_copy(ones_v, dacc.at[dst_v.at[g]], add=True)

        def oloop(o, carry):
            pltpu.sync_copy(src_hbm.at[pl.ds(gb + o * IC, IC), :], src_v)
            pltpu.sync_copy(dst_hbm.at[pl.ds(gb + o * IC, IC), :], dst_v)
            for b in range(3):
                fire(b, b)

            def eloop(j, c2):
                g4 = 4 * j
                for b in range(4):
                    g = g4 + b

                    @pl.when(g + 3 < IC)
                    def _(b=b, g=g):
                        fire((b + 3) % 4, g + 3)
                    wait(b, g)
                    scat(b, g)
                return c2
            lax.fori_loop(0, IC // 4, eloop, 0)
            return carry
        lax.fori_loop(0, nout, oloop, 0)
        plsc.subcore_barrier()

        pltpu.sync_copy(acc.at[pl.ds(base, RPS), :],
                        agg_hbm.at[cid, pl.ds(base, RPS), :])
        if with_deg:
            pltpu.sync_copy(dacc.at[pl.ds(base, RPS)],
                            deg_hbm.at[cid, pl.ds(base, RPS)])

    return pl.kernel(body, out_type=tuple(out_type), mesh=mesh,
                     scratch_types=scratch)


_SC_AGG_DEG = _build_sc_agg(True)
_SC_AGG = _build_sc_agg(False)

_TCB = 1024


def _tc_layer(x_pad, aggp, degp3, w_s, w_n, b):
    def body(x_ref, a_ref, d_ref, ws_ref, wn_ref, b_ref, o_ref):
        dg = jnp.maximum(d_ref[0] + d_ref[1], 1.0)
        agg = (a_ref[0] + a_ref[1]) / dg
        h = jnp.dot(x_ref[...], ws_ref[...], preferred_element_type=jnp.float32)
        h = h + jnp.dot(agg, wn_ref[...], preferred_element_type=jnp.float32)
        o_ref[...] = jnp.maximum(h + b_ref[...], 0.0)

    return pl.pallas_call(
        body,
        grid=(N_PAD // _TCB,),
        in_specs=[
            pl.BlockSpec((_TCB, D), lambda i: (i, 0)),
            pl.BlockSpec((2, _TCB, D), lambda i: (0, i, 0)),
            pl.BlockSpec((2, _TCB, 1), lambda i: (0, i, 0)),
            pl.BlockSpec((D, D), lambda i: (0, 0)),
            pl.BlockSpec((D, D), lambda i: (0, 0)),
            pl.BlockSpec((1, D), lambda i: (0, 0)),
        ],
        out_specs=pl.BlockSpec((_TCB, D), lambda i: (i, 0)),
        out_shape=jax.ShapeDtypeStruct((N_PAD, D), jnp.float32),
    )(x_pad, aggp, degp3, w_s, w_n, b)


def _tc_layer_head(h1, aggp, degp3, w_s, w_n, b, w_out, b_out):
    def body(x_ref, a_ref, d_ref, ws_ref, wn_ref, b_ref, wo_ref, bo_ref, o_ref):
        dg = jnp.maximum(d_ref[0] + d_ref[1], 1.0)
        agg = (a_ref[0] + a_ref[1]) / dg
        h = jnp.dot(x_ref[...], ws_ref[...], preferred_element_type=jnp.float32)
        h = h + jnp.dot(agg, wn_ref[...], preferred_element_type=jnp.float32)
        h2 = jnp.maximum(h + b_ref[...], 0.0)
        y = jnp.dot(h2, wo_ref[...], preferred_element_type=jnp.float32)
        z = (y + bo_ref[...] - MU) / SIGMA
        o_ref[...] = jnp.exp(-0.5 * z * z)

    return pl.pallas_call(
        body,
        grid=(N_PAD // _TCB,),
        in_specs=[
            pl.BlockSpec((_TCB, D), lambda i: (i, 0)),
            pl.BlockSpec((2, _TCB, D), lambda i: (0, i, 0)),
            pl.BlockSpec((2, _TCB, 1), lambda i: (0, i, 0)),
            pl.BlockSpec((D, D), lambda i: (0, 0)),
            pl.BlockSpec((D, D), lambda i: (0, 0)),
            pl.BlockSpec((1, D), lambda i: (0, 0)),
            pl.BlockSpec((D, HW), lambda i: (0, 0)),
            pl.BlockSpec((1, HW), lambda i: (0, 0)),
        ],
        out_specs=pl.BlockSpec((_TCB, HW), lambda i: (i, 0)),
        out_shape=jax.ShapeDtypeStruct((N_PAD, HW), jnp.float32),
    )(h1, aggp, degp3, w_s, w_n, b, w_out, b_out)


def kernel(x, edge_index, W1_self, W1_nbr, b1, W2_self, W2_nbr, b2, W_out, b_out):
    src = edge_index[0]
    dst = edge_index[1]
    pad_e = E_PAD - E
    srcr = jnp.concatenate(
        [src, jnp.zeros((pad_e,), jnp.int32)]).reshape(E_PAD // G, G)
    dstr = jnp.concatenate(
        [dst, jnp.full((pad_e,), N_PAD - 1, jnp.int32)]).reshape(E_PAD // G, G)
    x_pad = jnp.pad(x, ((0, N_PAD - N), (0, 0)))

    agg1p, degp = _SC_AGG_DEG(x_pad, srcr, dstr)
    degp3 = degp.reshape(2, N_PAD, 1)
    h1 = _tc_layer(x_pad, agg1p, degp3, W1_self, W1_nbr, b1.reshape(1, D))
    (agg2p,) = _SC_AGG(h1, srcr, dstr)
    out_w = _tc_layer_head(
        h1, agg2p, degp3, W2_self, W2_nbr, b2.reshape(1, D),
        jnp.pad(W_out, ((0, 0), (0, HW - 1))),
        jnp.pad(b_out, (0, HW - 1)).reshape(1, HW))
    return out_w[:N, :1]
